# Initial kernel scaffold; baseline (speedup 1.0000x reference)
import jax
import jax.numpy as jnp
from jax.experimental import pallas as pl


def kernel(features, edge_index, W1, b1, W2, b2, W3, b3):
    raise NotImplementedError("write your pallas kernel here")


# speedup vs baseline: 6.9745x; 6.9745x over previous
import functools
import jax
import jax.numpy as jnp
from jax import lax
from jax.experimental import pallas as pl
from jax.experimental.pallas import tpu as pltpu
from jax.experimental.pallas import tpu_sc as plsc

NN = 10000
EE = 320000
DD = 128
HH = 16
CC = 40

NCORES = 2
NSUB = 16
NW = NCORES * NSUB

CHUNK = 128
EPT = ((EE + NW * CHUNK - 1) // (NW * CHUNK)) * CHUNK
EPAD = EPT * NW
NCHUNK = EPT // CHUNK

DEG_ROWS = 20480
AGG_ROWS = 10240

_MESH = plsc.VectorSubcoreMesh(core_axis_name="c", subcore_axis_name="s")
_SC_PARAMS = pltpu.CompilerParams(use_tc_tiling_on_sc=False)


def _fill_rows(buf, nrows, value):
    def body(i, _):
        buf[i, :] = jnp.full((HH,), value, jnp.float32)
        return 0
    lax.fori_loop(0, nrows, body, 0)


def _zero_table(table, zbuf, sub, rows):
    per = rows // NSUB
    base = sub * per
    def body(i, _):
        pltpu.sync_copy(zbuf, table.at[pl.ds(base + i * CHUNK, CHUNK)])
        return 0
    lax.fori_loop(0, per // CHUNK, body, 0)


@functools.partial(
    pl.kernel,
    out_type=jax.ShapeDtypeStruct((NCORES, DEG_ROWS, HH), jnp.float32),
    mesh=_MESH,
    compiler_params=_SC_PARAMS,
    scratch_types=[
        pltpu.VMEM((CHUNK, HH), jnp.float32),
        pltpu.VMEM((CHUNK, HH), jnp.float32),
        pltpu.VMEM((CHUNK,), jnp.int32),
        pltpu.VMEM((CHUNK,), jnp.int32),
        pltpu.VMEM_SHARED((DEG_ROWS, HH), jnp.float32),
    ],
)
def _deg_kernel(srcd, dstd, out, ones_v, zeros_v, sidx, didx, table):
    c = lax.axis_index("c")
    s = lax.axis_index("s")
    wid = s * NCORES + c

    _fill_rows(ones_v, CHUNK, 1.0)
    _fill_rows(zeros_v, CHUNK, 0.0)
    _zero_table(table, zeros_v, s, DEG_ROWS)
    plsc.subcore_barrier()

    def step(i, _):
        off = wid * EPT + i * CHUNK
        pltpu.sync_copy(srcd.at[pl.ds(off, CHUNK)], sidx)
        pltpu.sync_copy(dstd.at[pl.ds(off, CHUNK)], didx)
        pltpu.sync_copy(ones_v, table.at[sidx], add=True)
        pltpu.sync_copy(ones_v, table.at[didx], add=True)
        return 0
    lax.fori_loop(0, NCHUNK, step, 0)

    plsc.subcore_barrier()
    per = DEG_ROWS // NSUB
    pltpu.sync_copy(table.at[pl.ds(s * per, per)],
                    out.at[c, pl.ds(s * per, per)])


@functools.partial(
    pl.kernel,
    out_type=jax.ShapeDtypeStruct((NCORES, AGG_ROWS, HH), jnp.float32),
    mesh=_MESH,
    compiler_params=_SC_PARAMS,
    scratch_types=[
        pltpu.VMEM((CHUNK, HH), jnp.float32),
        pltpu.VMEM((CHUNK, HH), jnp.float32),
        pltpu.VMEM((CHUNK,), jnp.int32),
        pltpu.VMEM((CHUNK,), jnp.int32),
        pltpu.VMEM_SHARED((AGG_ROWS, HH), jnp.float32),
        pltpu.SemaphoreType.DMA,
    ],
)
def _agg_kernel(y, srcp, dstp, out, rows_v, zeros_v, sidx, didx, table, sem):
    c = lax.axis_index("c")
    s = lax.axis_index("s")
    wid = s * NCORES + c

    _fill_rows(zeros_v, CHUNK, 0.0)
    _zero_table(table, zeros_v, s, AGG_ROWS)
    plsc.subcore_barrier()

    def step(i, _):
        off = wid * EPT + i * CHUNK
        pltpu.sync_copy(srcp.at[pl.ds(off, CHUNK)], sidx)
        pltpu.sync_copy(dstp.at[pl.ds(off, CHUNK)], didx)
        pltpu.async_copy(y.at[sidx], rows_v, sem).wait()
        pltpu.sync_copy(rows_v, table.at[didx], add=True)
        return 0
    lax.fori_loop(0, NCHUNK, step, 0)

    plsc.subcore_barrier()
    per = AGG_ROWS // NSUB
    pltpu.sync_copy(table.at[pl.ds(s * per, per)],
                    out.at[c, pl.ds(s * per, per)])


def _dense1_body(x_ref, w_ref, z_ref):
    z_ref[...] = jnp.dot(x_ref[...], w_ref[...],
                         preferred_element_type=jnp.float32)


def _norm_body(deg_ref, z_ref, y_ref, ns_ref, nd_ref):
    dsum = deg_ref[0] + deg_ref[1]
    deg_out = dsum[:NN, 0:1]
    deg_in = dsum[NN:2 * NN, 0:1]
    ns = jnp.where(deg_out > 0, lax.rsqrt(jnp.maximum(deg_out, 1e-12)), 0.0)
    nd = jnp.where(deg_in > 0, lax.rsqrt(jnp.maximum(deg_in, 1e-12)), 0.0)
    ns_ref[...] = ns
    nd_ref[...] = nd
    y_ref[...] = z_ref[...] * ns


def _mid_body(agg_ref, ns_ref, nd_ref, b_ref, w_ref, y_ref):
    aggsum = agg_ref[0, :NN] + agg_ref[1, :NN]
    h = jnp.maximum(aggsum * nd_ref[...] + b_ref[...], 0.0)
    y_ref[...] = jnp.dot(h * ns_ref[...], w_ref[...],
                         preferred_element_type=jnp.float32)


def _premix_body(agg_ref, ns_ref, nd_ref, b_ref, y_ref):
    aggsum = agg_ref[0, :NN] + agg_ref[1, :NN]
    h = jnp.maximum(aggsum * nd_ref[...] + b_ref[...], 0.0)
    y_ref[...] = h * ns_ref[...]


def _final_body(agg_ref, nd_ref, b_ref, w_ref, out_ref):
    aggsum = agg_ref[0, :NN] + agg_ref[1, :NN]
    out_ref[...] = jnp.dot(aggsum * nd_ref[...], w_ref[...],
                           preferred_element_type=jnp.float32) + b_ref[...]


def _tc(body, out_shape):
    return pl.pallas_call(body, out_shape=out_shape)


_f32 = jnp.float32


def kernel(features, edge_index, W1, b1, W2, b2, W3, b3):
    src = edge_index[0]
    dst = edge_index[1]
    npad = EPAD - EE
    srcp = jnp.concatenate([src, jnp.zeros((npad,), jnp.int32)])
    dstp = jnp.concatenate([dst, jnp.full((npad,), NN, jnp.int32)])
    srcd = jnp.concatenate([src, jnp.full((npad,), 2 * NN, jnp.int32)])
    dstd = jnp.concatenate([dst + NN, jnp.full((npad,), 2 * NN, jnp.int32)])

    deg_parts = _deg_kernel(srcd, dstd)
    z1 = _tc(_dense1_body, jax.ShapeDtypeStruct((NN, HH), _f32))(features, W1)
    y1, ns, nd = _tc(_norm_body, (jax.ShapeDtypeStruct((NN, HH), _f32),
                                  jax.ShapeDtypeStruct((NN, 1), _f32),
                                  jax.ShapeDtypeStruct((NN, 1), _f32)))(
        deg_parts, z1)

    a1 = _agg_kernel(y1, srcp, dstp)
    y2 = _tc(_mid_body, jax.ShapeDtypeStruct((NN, HH), _f32))(
        a1, ns, nd, b1.reshape(1, HH), W2)

    a2 = _agg_kernel(y2, srcp, dstp)
    y3 = _tc(_premix_body, jax.ShapeDtypeStruct((NN, HH), _f32))(
        a2, ns, nd, b2.reshape(1, HH))

    a3 = _agg_kernel(y3, srcp, dstp)
    out = _tc(_final_body, jax.ShapeDtypeStruct((NN, CC), _f32))(
        a3, nd, b3.reshape(1, CC), W3)
    return out


# speedup vs baseline: 11.9753x; 1.7170x over previous
import functools
import jax
import jax.numpy as jnp
from jax import lax
from jax.experimental import pallas as pl
from jax.experimental.pallas import tpu as pltpu
from jax.experimental.pallas import tpu_sc as plsc

NN = 10000
EE = 320000
DD = 128
HH = 16
CC = 40

NCORES = 2
NSUB = 16
NW = NCORES * NSUB

CHUNK = 128
RING = 8
EPT = 10240
EPAD = EPT * NW
NCHUNK = EPT // CHUNK
NG = NCHUNK // RING

DEG_ROWS = 20480
AGG_ROWS = 10240

_MESH = plsc.VectorSubcoreMesh(core_axis_name="c", subcore_axis_name="s")
_SC_PARAMS = pltpu.CompilerParams(use_tc_tiling_on_sc=False)


def _fill_rows(buf, nrows, value):
    def body(i, _):
        buf[i, :] = jnp.full((HH,), value, jnp.float32)
        return 0
    lax.fori_loop(0, nrows, body, 0)


def _zero_table(table, zbuf, sub, rows, sem):
    per = rows // NSUB
    base = sub * per
    descs = [
        pltpu.async_copy(zbuf, table.at[pl.ds(base + i * CHUNK, CHUNK)], sem)
        for i in range(per // CHUNK)
    ]
    for d in descs:
        d.wait()


@functools.partial(
    pl.kernel,
    out_type=jax.ShapeDtypeStruct((NCORES, DEG_ROWS, HH), jnp.float32),
    mesh=_MESH,
    compiler_params=_SC_PARAMS,
    scratch_types=[
        pltpu.VMEM((CHUNK, HH), jnp.float32),
        pltpu.VMEM((CHUNK, HH), jnp.float32),
        pltpu.VMEM((2 * NCHUNK, CHUNK), jnp.int32),
        pltpu.VMEM_SHARED((DEG_ROWS, HH), jnp.float32),
        pltpu.SemaphoreType.DMA,
    ],
)
def _deg_kernel(idx3, out, ones_v, zeros_v, idx_v, table, sem0):
    c = lax.axis_index("c")
    s = lax.axis_index("s")
    wid = s * NCORES + c

    pltpu.sync_copy(idx3.at[wid], idx_v)
    _fill_rows(ones_v, CHUNK, 1.0)
    _fill_rows(zeros_v, CHUNK, 0.0)
    _zero_table(table, zeros_v, s, DEG_ROWS, sem0)
    plsc.subcore_barrier()

    @pl.loop(0, 2 * NG)
    def _(g):
        descs = [
            pltpu.async_copy(ones_v, table.at[idx_v.at[g * RING + b]],
                             sem0, add=True)
            for b in range(RING)
        ]
        for d in descs:
            d.wait()

    plsc.subcore_barrier()
    per = DEG_ROWS // NSUB
    pltpu.sync_copy(table.at[pl.ds(s * per, per)],
                    out.at[c, pl.ds(s * per, per)])


@functools.partial(
    pl.kernel,
    out_type=jax.ShapeDtypeStruct((NCORES, AGG_ROWS, HH), jnp.float32),
    mesh=_MESH,
    compiler_params=_SC_PARAMS,
    scratch_types=[
        pltpu.VMEM((NCHUNK, CHUNK), jnp.int32),
        pltpu.VMEM((NCHUNK, CHUNK), jnp.int32),
        pltpu.VMEM((2, RING, CHUNK, HH), jnp.float32),
        pltpu.VMEM((CHUNK, HH), jnp.float32),
        pltpu.VMEM_SHARED((AGG_ROWS, HH), jnp.float32),
        pltpu.SemaphoreType.DMA,
        pltpu.SemaphoreType.DMA,
    ],
)
def _agg_kernel(y, srcp3, dstp3, out, sidx, didx, rows, zeros_v, table,
                gsem, ssem):
    c = lax.axis_index("c")
    s = lax.axis_index("s")
    wid = s * NCORES + c

    pltpu.sync_copy(srcp3.at[wid], sidx)
    pltpu.sync_copy(dstp3.at[wid], didx)
    _fill_rows(zeros_v, CHUNK, 0.0)
    _zero_table(table, zeros_v, s, AGG_ROWS, gsem)
    plsc.subcore_barrier()

    @pl.loop(0, NG)
    def _(g):
        gd = [
            pltpu.async_copy(y.at[sidx.at[g * RING + b]], rows.at[0, b], gsem)
            for b in range(RING)
        ]
        for d in gd:
            d.wait()
        sd = [
            pltpu.async_copy(rows.at[0, b], table.at[didx.at[g * RING + b]],
                             ssem, add=True)
            for b in range(RING)
        ]
        for d in sd:
            d.wait()

    plsc.subcore_barrier()
    per = AGG_ROWS // NSUB
    pltpu.sync_copy(table.at[pl.ds(s * per, per)],
                    out.at[c, pl.ds(s * per, per)])


def _dense1_body(x_ref, w_ref, z_ref):
    z_ref[...] = jnp.dot(x_ref[...], w_ref[...],
                         preferred_element_type=jnp.float32)


def _norm_body(deg_ref, z_ref, y_ref, ns_ref, nd_ref):
    dsum = deg_ref[0] + deg_ref[1]
    deg_out = dsum[:NN, 0:1]
    deg_in = dsum[NN:2 * NN, 0:1]
    ns = jnp.where(deg_out > 0, lax.rsqrt(jnp.maximum(deg_out, 1e-12)), 0.0)
    nd = jnp.where(deg_in > 0, lax.rsqrt(jnp.maximum(deg_in, 1e-12)), 0.0)
    ns_ref[...] = ns
    nd_ref[...] = nd
    y_ref[...] = z_ref[...] * ns


def _mid_body(agg_ref, ns_ref, nd_ref, b_ref, w_ref, y_ref):
    aggsum = agg_ref[0, :NN] + agg_ref[1, :NN]
    h = jnp.maximum(aggsum * nd_ref[...] + b_ref[...], 0.0)
    y_ref[...] = jnp.dot(h * ns_ref[...], w_ref[...],
                         preferred_element_type=jnp.float32)


def _premix_body(agg_ref, ns_ref, nd_ref, b_ref, y_ref):
    aggsum = agg_ref[0, :NN] + agg_ref[1, :NN]
    h = jnp.maximum(aggsum * nd_ref[...] + b_ref[...], 0.0)
    y_ref[...] = h * ns_ref[...]


def _final_body(agg_ref, nd_ref, b_ref, w_ref, out_ref):
    aggsum = agg_ref[0, :NN] + agg_ref[1, :NN]
    out_ref[...] = jnp.dot(aggsum * nd_ref[...], w_ref[...],
                           preferred_element_type=jnp.float32) + b_ref[...]


def _tc(body, out_shape):
    return pl.pallas_call(body, out_shape=out_shape)


_f32 = jnp.float32


def kernel(features, edge_index, W1, b1, W2, b2, W3, b3):
    src = edge_index[0]
    dst = edge_index[1]
    npad = EPAD - EE
    srcp = jnp.concatenate([src, jnp.zeros((npad,), jnp.int32)])
    dstp = jnp.concatenate([dst, jnp.full((npad,), NN, jnp.int32)])
    srcd = jnp.concatenate([src, jnp.full((npad,), 2 * NN, jnp.int32)])
    dstd = jnp.concatenate([dst + NN, jnp.full((npad,), 2 * NN, jnp.int32)])
    srcp3 = srcp.reshape(NW, NCHUNK, CHUNK)
    dstp3 = dstp.reshape(NW, NCHUNK, CHUNK)
    idx3 = jnp.concatenate([srcd.reshape(NW, NCHUNK, CHUNK),
                            dstd.reshape(NW, NCHUNK, CHUNK)], axis=1)

    deg_parts = _deg_kernel(idx3)
    z1 = _tc(_dense1_body, jax.ShapeDtypeStruct((NN, HH), _f32))(features, W1)
    y1, ns, nd = _tc(_norm_body, (jax.ShapeDtypeStruct((NN, HH), _f32),
                                  jax.ShapeDtypeStruct((NN, 1), _f32),
                                  jax.ShapeDtypeStruct((NN, 1), _f32)))(
        deg_parts, z1)

    a1 = _agg_kernel(y1, srcp3, dstp3)
    y2 = _tc(_mid_body, jax.ShapeDtypeStruct((NN, HH), _f32))(
        a1, ns, nd, b1.reshape(1, HH), W2)

    a2 = _agg_kernel(y2, srcp3, dstp3)
    y3 = _tc(_premix_body, jax.ShapeDtypeStruct((NN, HH), _f32))(
        a2, ns, nd, b2.reshape(1, HH))

    a3 = _agg_kernel(y3, srcp3, dstp3)
    out = _tc(_final_body, jax.ShapeDtypeStruct((NN, CC), _f32))(
        a3, nd, b3.reshape(1, CC), W3)
    return out
